# Initial kernel scaffold; baseline (speedup 1.0000x reference)
import functools

import jax
import jax.numpy as jnp
from jax import lax
from jax.experimental import pallas as pl
from jax.experimental.pallas import tpu as pltpu
from jax.experimental.pallas import tpu_sc as plsc

N = 10000
E = 320000
D = 64
NC = 2
NS = 16
NW = NC * NS
EPT = E // NW
NPT = N // NS


def _mesh():
    return plsc.VectorSubcoreMesh(core_axis_name="c", subcore_axis_name="s")


def _zero_rows(buf, nrows):
    zero = jnp.zeros((16,), jnp.float32)

    def body(i, _):
        for j in range(D // 16):
            buf[i, pl.ds(j * 16, 16)] = zero
        return 0

    lax.fori_loop(0, nrows, body, 0, unroll=False)


CB_A = 1000


def _segsum_body(eh_hbm, src_hbm, out_hbm, rows_v, idx_v, acc_sh, sem):
    c = lax.axis_index("c")
    s = lax.axis_index("s")

    _zero_rows(rows_v, NPT)
    row0 = s * NPT
    pltpu.sync_copy(rows_v.at[pl.ds(0, NPT)], acc_sh.at[pl.ds(row0, NPT)])
    plsc.subcore_barrier()

    base0 = (c * NS + s) * EPT
    for k in range(EPT // CB_A):
        base = base0 + k * CB_A
        pltpu.sync_copy(src_hbm.at[pl.ds(base, CB_A)], idx_v)
        pltpu.sync_copy(eh_hbm.at[pl.ds(base, CB_A)], rows_v)
        pltpu.sync_copy(rows_v, acc_sh.at[idx_v], add=True)

    plsc.subcore_barrier()
    pltpu.sync_copy(acc_sh.at[pl.ds(row0, NPT)], out_hbm.at[c, pl.ds(row0, NPT)])


def _segsum_eh(edge_hidden, src):
    k = functools.partial(
        pl.kernel,
        out_type=jax.ShapeDtypeStruct((NC, N, D), jnp.float32),
        mesh=_mesh(),
        scratch_types=[
            pltpu.VMEM((CB_A, D), jnp.float32),
            pltpu.VMEM((CB_A,), jnp.int32),
            pltpu.VMEM_SHARED((N, D), jnp.float32),
            pltpu.SemaphoreType.DMA,
        ],
    )(_segsum_body)
    return k(edge_hidden, src)


H_A, H_B = 504, 496


def _msg_body(y1_hbm, z_hbm, a2_hbm, src_hbm, dst_hbm, out_hbm,
              sa_v, sb_v, da_v, db_v, ys_v, zd_v, a2_v, acc_sh, sem):
    c = lax.axis_index("c")
    s = lax.axis_index("s")

    _zero_rows(ys_v, H_A)
    row0 = s * NPT
    pltpu.sync_copy(ys_v.at[pl.ds(0, H_A)], acc_sh.at[pl.ds(row0, H_A)])
    pltpu.sync_copy(ys_v.at[pl.ds(0, NPT - H_A)],
                    acc_sh.at[pl.ds(row0 + H_A, NPT - H_A)])
    plsc.subcore_barrier()

    def compute(nrows):
        def body(i, _):
            for j in range(D // 16):
                sl = pl.ds(j * 16, 16)
                v = a2_v[i, sl] + ys_v[i, sl] + zd_v[i, sl]
                a2_v[i, sl] = jnp.maximum(v, 0.0)
            return 0
        lax.fori_loop(0, nrows, body, 0, unroll=False)

    base0 = (c * NS + s) * EPT
    for k in range(EPT // (H_A + H_B)):
        blk = base0 + k * (H_A + H_B)
        for (off, h, idx_s, idx_d) in ((0, H_A, sa_v, da_v),
                                       (H_A, H_B, sb_v, db_v)):
            base = blk + off
            pltpu.sync_copy(src_hbm.at[pl.ds(base, h)], idx_s)
            pltpu.sync_copy(dst_hbm.at[pl.ds(base, h)], idx_d)
            pltpu.async_copy(y1_hbm.at[idx_s], ys_v.at[pl.ds(0, h)], sem).wait()
            pltpu.async_copy(z_hbm.at[idx_d], zd_v.at[pl.ds(0, h)], sem).wait()
            pltpu.sync_copy(a2_hbm.at[pl.ds(base, h)], a2_v.at[pl.ds(0, h)])
            compute(h)
            pltpu.sync_copy(a2_v.at[pl.ds(0, h)], acc_sh.at[idx_s], add=True)

    plsc.subcore_barrier()
    pltpu.sync_copy(acc_sh.at[pl.ds(row0, NPT)], out_hbm.at[c, pl.ds(row0, NPT)])


def _msg_segsum(y1, z, a2, src, dst):
    k = functools.partial(
        pl.kernel,
        out_type=jax.ShapeDtypeStruct((NC, N, D), jnp.float32),
        mesh=_mesh(),
        scratch_types=[
            pltpu.VMEM((H_A,), jnp.int32),
            pltpu.VMEM((H_B,), jnp.int32),
            pltpu.VMEM((H_A,), jnp.int32),
            pltpu.VMEM((H_B,), jnp.int32),
            pltpu.VMEM((H_A, D), jnp.float32),
            pltpu.VMEM((H_A, D), jnp.float32),
            pltpu.VMEM((H_A, D), jnp.float32),
            pltpu.VMEM_SHARED((N, D), jnp.float32),
            pltpu.SemaphoreType.DMA,
        ],
    )(_msg_body)
    return k(y1, z, a2, src, dst)


_DN = (((1,), (1,)), ((), ()))


def _node_mm_kernel(x_ref, s2_ref, w1_ref, w3_ref, u1_ref, y1_ref, z_ref, yu_ref):
    xb = x_ref[...]
    y1_ref[...] = lax.dot_general(xb, w1_ref[...], _DN,
                                  preferred_element_type=jnp.float32)
    yu_ref[...] = lax.dot_general(xb, u1_ref[...], _DN,
                                  preferred_element_type=jnp.float32)
    sb = s2_ref[0] + s2_ref[1]
    z_ref[...] = lax.dot_general(sb, w3_ref[...], _DN,
                                 preferred_element_type=jnp.float32)


def _node_mm(x, s2, w1, w3, u1):
    bn = 1000
    grid = (N // bn,)
    return pl.pallas_call(
        _node_mm_kernel,
        grid=grid,
        in_specs=[
            pl.BlockSpec((bn, 128), lambda i: (i, 0)),
            pl.BlockSpec((NC, bn, D), lambda i: (0, i, 0)),
            pl.BlockSpec((D, 128), lambda i: (0, 0)),
            pl.BlockSpec((D, D), lambda i: (0, 0)),
            pl.BlockSpec((D, 128), lambda i: (0, 0)),
        ],
        out_specs=[
            pl.BlockSpec((bn, D), lambda i: (i, 0)),
            pl.BlockSpec((bn, D), lambda i: (i, 0)),
            pl.BlockSpec((bn, D), lambda i: (i, 0)),
        ],
        out_shape=[jax.ShapeDtypeStruct((N, D), jnp.float32)] * 3,
    )(x, s2, w1, w3, u1)


def _edge_mm_kernel(ea_ref, w2_ref, a2_ref):
    a2_ref[...] = lax.dot_general(ea_ref[...], w2_ref[...], _DN,
                                  preferred_element_type=jnp.float32)


def _edge_mm(edge_attr, w2):
    be = 8000
    grid = (E // be,)
    return pl.pallas_call(
        _edge_mm_kernel,
        grid=grid,
        in_specs=[
            pl.BlockSpec((be, 16), lambda i: (i, 0)),
            pl.BlockSpec((D, 16), lambda i: (0, 0)),
        ],
        out_specs=pl.BlockSpec((be, D), lambda i: (i, 0)),
        out_shape=jax.ShapeDtypeStruct((E, D), jnp.float32),
    )(edge_attr, w2)


def _final_mm_kernel(yu_ref, p2_ref, u2_ref, out_ref):
    pb = p2_ref[0] + p2_ref[1]
    out_ref[...] = yu_ref[...] + lax.dot_general(
        pb, u2_ref[...], _DN, preferred_element_type=jnp.float32)


def _final_mm(yu, p2, u2):
    bn = 1000
    grid = (N // bn,)
    return pl.pallas_call(
        _final_mm_kernel,
        grid=grid,
        in_specs=[
            pl.BlockSpec((bn, D), lambda i: (i, 0)),
            pl.BlockSpec((NC, bn, D), lambda i: (0, i, 0)),
            pl.BlockSpec((D, D), lambda i: (0, 0)),
        ],
        out_specs=pl.BlockSpec((bn, D), lambda i: (i, 0)),
        out_shape=jax.ShapeDtypeStruct((N, D), jnp.float32),
    )(yu, p2, u2)


def kernel(x, edge_attr, edge_hidden, W1, W2, W3, U1, U2, edge_index):
    src = edge_index[0]
    dst = edge_index[1]

    s2 = _segsum_eh(edge_hidden, src)
    a2 = _edge_mm(edge_attr, W2)
    y1, z, yu = _node_mm(x, s2, W1, W3, U1)
    p2 = _msg_segsum(y1, z, a2, src, dst)
    return _final_mm(yu, p2, U2)


# speedup vs baseline: 3.0157x; 3.0157x over previous
import functools

import jax
import jax.numpy as jnp
from jax import lax
from jax.experimental import pallas as pl
from jax.experimental.pallas import tpu as pltpu
from jax.experimental.pallas import tpu_sc as plsc

N = 10000
E = 320000
D = 64
NC = 2
NS = 16
NW = NC * NS
EPT = E // NW
NPT_BASE = 624
NPT_LAST = N - (NS - 1) * NPT_BASE


def _mesh():
    return plsc.VectorSubcoreMesh(core_axis_name="c", subcore_axis_name="s")


def _zero_rows(buf, nrows):
    zero = jnp.zeros((16,), jnp.float32)

    def body(i, _):
        for j in range(D // 16):
            buf[i, pl.ds(j * 16, 16)] = zero
        return 0

    lax.fori_loop(0, nrows, body, 0, unroll=False)


def _zero_acc_slab(buf, acc_sh, s, cap):
    row0 = s * NPT_BASE
    _zero_rows(buf, min(cap, NPT_LAST))

    def copy_rows(total):
        off = 0
        while off < total:
            h = min(cap, total - off)
            pltpu.sync_copy(buf.at[pl.ds(0, h)], acc_sh.at[pl.ds(row0 + off, h)])
            off += h

    @pl.when(s == NS - 1)
    def _():
        copy_rows(NPT_LAST)

    @pl.when(s != NS - 1)
    def _():
        copy_rows(NPT_BASE)


def _dump_acc_slab(acc_sh, out_hbm, buf, c, s, cap):
    row0 = s * NPT_BASE

    def copy_rows(total):
        off = 0
        while off < total:
            h = min(cap, total - off)
            pltpu.sync_copy(acc_sh.at[pl.ds(row0 + off, h)], buf.at[pl.ds(0, h)])
            pltpu.sync_copy(buf.at[pl.ds(0, h)], out_hbm.at[c, pl.ds(row0 + off, h)])
            off += h

    @pl.when(s == NS - 1)
    def _():
        copy_rows(NPT_LAST)

    @pl.when(s != NS - 1)
    def _():
        copy_rows(NPT_BASE)


CB_A = 400


def _segsum_body(eh_hbm, src_hbm, out_hbm, rows_v, idx_v, acc_sh, sem):
    c = lax.axis_index("c")
    s = lax.axis_index("s")

    _zero_acc_slab(rows_v, acc_sh, s, CB_A)
    plsc.subcore_barrier()

    base0 = (c * NS + s) * EPT
    for k in range(EPT // CB_A):
        base = base0 + k * CB_A
        pltpu.sync_copy(src_hbm.at[pl.ds(base, CB_A)], idx_v)
        pltpu.sync_copy(eh_hbm.at[pl.ds(base, CB_A)], rows_v)
        pltpu.sync_copy(rows_v, acc_sh.at[idx_v], add=True)

    plsc.subcore_barrier()
    _dump_acc_slab(acc_sh, out_hbm, rows_v, c, s, CB_A)


def _segsum_eh(edge_hidden, src):
    k = functools.partial(
        pl.kernel,
        out_type=jax.ShapeDtypeStruct((NC, N, D), jnp.float32),
        mesh=_mesh(),
        compiler_params=pltpu.CompilerParams(use_tc_tiling_on_sc=False),
        scratch_types=[
            pltpu.VMEM((CB_A, D), jnp.float32),
            pltpu.VMEM((CB_A,), jnp.int32),
            pltpu.VMEM_SHARED((N, D), jnp.float32),
            pltpu.SemaphoreType.DMA,
        ],
    )(_segsum_body)
    return k(edge_hidden, src)


CB_C = 400


def _msg_body(y1_hbm, z_hbm, a2_hbm, src_hbm, dst_hbm, out_hbm,
              src_v, dst_v, ys_v, zd_v, a2_v, acc_sh, sem):
    c = lax.axis_index("c")
    s = lax.axis_index("s")

    _zero_acc_slab(ys_v, acc_sh, s, CB_C)
    plsc.subcore_barrier()

    def compute():
        def body(i, _):
            for j in range(D // 16):
                sl = pl.ds(j * 16, 16)
                v = a2_v[i, sl] + ys_v[i, sl] + zd_v[i, sl]
                a2_v[i, sl] = jnp.maximum(v, 0.0)
            return 0
        lax.fori_loop(0, CB_C, body, 0, unroll=False)

    base0 = (c * NS + s) * EPT
    for k in range(EPT // CB_C):
        base = base0 + k * CB_C
        pltpu.sync_copy(src_hbm.at[pl.ds(base, CB_C)], src_v)
        pltpu.sync_copy(dst_hbm.at[pl.ds(base, CB_C)], dst_v)
        pltpu.async_copy(y1_hbm.at[src_v], ys_v, sem).wait()
        pltpu.async_copy(z_hbm.at[dst_v], zd_v, sem).wait()
        pltpu.sync_copy(a2_hbm.at[pl.ds(base, CB_C)], a2_v)
        compute()
        pltpu.sync_copy(a2_v, acc_sh.at[src_v], add=True)

    plsc.subcore_barrier()
    _dump_acc_slab(acc_sh, out_hbm, ys_v, c, s, CB_C)


def _msg_segsum(y1, z, a2, src, dst):
    k = functools.partial(
        pl.kernel,
        out_type=jax.ShapeDtypeStruct((NC, N, D), jnp.float32),
        mesh=_mesh(),
        compiler_params=pltpu.CompilerParams(use_tc_tiling_on_sc=False),
        scratch_types=[
            pltpu.VMEM((CB_C,), jnp.int32),
            pltpu.VMEM((CB_C,), jnp.int32),
            pltpu.VMEM((CB_C, D), jnp.float32),
            pltpu.VMEM((CB_C, D), jnp.float32),
            pltpu.VMEM((CB_C, D), jnp.float32),
            pltpu.VMEM_SHARED((N, D), jnp.float32),
            pltpu.SemaphoreType.DMA,
        ],
    )(_msg_body)
    return k(y1, z, a2, src, dst)


_DN = (((1,), (1,)), ((), ()))


def _node_mm_kernel(x_ref, s2_ref, w1_ref, w3_ref, u1_ref, y1_ref, z_ref, yu_ref):
    xb = x_ref[...]
    y1_ref[...] = lax.dot_general(xb, w1_ref[...], _DN,
                                  preferred_element_type=jnp.float32)
    yu_ref[...] = lax.dot_general(xb, u1_ref[...], _DN,
                                  preferred_element_type=jnp.float32)
    sb = s2_ref[0] + s2_ref[1]
    z_ref[...] = lax.dot_general(sb, w3_ref[...], _DN,
                                 preferred_element_type=jnp.float32)


def _node_mm(x, s2, w1, w3, u1):
    bn = 1000
    grid = (N // bn,)
    return pl.pallas_call(
        _node_mm_kernel,
        grid=grid,
        in_specs=[
            pl.BlockSpec((bn, 128), lambda i: (i, 0)),
            pl.BlockSpec((NC, bn, D), lambda i: (0, i, 0)),
            pl.BlockSpec((D, 128), lambda i: (0, 0)),
            pl.BlockSpec((D, D), lambda i: (0, 0)),
            pl.BlockSpec((D, 128), lambda i: (0, 0)),
        ],
        out_specs=[
            pl.BlockSpec((bn, D), lambda i: (i, 0)),
            pl.BlockSpec((bn, D), lambda i: (i, 0)),
            pl.BlockSpec((bn, D), lambda i: (i, 0)),
        ],
        out_shape=[jax.ShapeDtypeStruct((N, D), jnp.float32)] * 3,
    )(x, s2, w1, w3, u1)


def _edge_mm_kernel(ea_ref, w2_ref, a2_ref):
    a2_ref[...] = lax.dot_general(ea_ref[...], w2_ref[...], _DN,
                                  preferred_element_type=jnp.float32)


def _edge_mm(edge_attr, w2):
    be = 8000
    grid = (E // be,)
    return pl.pallas_call(
        _edge_mm_kernel,
        grid=grid,
        in_specs=[
            pl.BlockSpec((be, 16), lambda i: (i, 0)),
            pl.BlockSpec((D, 16), lambda i: (0, 0)),
        ],
        out_specs=pl.BlockSpec((be, D), lambda i: (i, 0)),
        out_shape=jax.ShapeDtypeStruct((E, D), jnp.float32),
    )(edge_attr, w2)


def _final_mm_kernel(yu_ref, p2_ref, u2_ref, out_ref):
    pb = p2_ref[0] + p2_ref[1]
    out_ref[...] = yu_ref[...] + lax.dot_general(
        pb, u2_ref[...], _DN, preferred_element_type=jnp.float32)


def _final_mm(yu, p2, u2):
    bn = 1000
    grid = (N // bn,)
    return pl.pallas_call(
        _final_mm_kernel,
        grid=grid,
        in_specs=[
            pl.BlockSpec((bn, D), lambda i: (i, 0)),
            pl.BlockSpec((NC, bn, D), lambda i: (0, i, 0)),
            pl.BlockSpec((D, D), lambda i: (0, 0)),
        ],
        out_specs=pl.BlockSpec((bn, D), lambda i: (i, 0)),
        out_shape=jax.ShapeDtypeStruct((N, D), jnp.float32),
    )(yu, p2, u2)


def kernel(x, edge_attr, edge_hidden, W1, W2, W3, U1, U2, edge_index):
    src = edge_index[0]
    dst = edge_index[1]

    s2 = _segsum_eh(edge_hidden, src)
    a2 = _edge_mm(edge_attr, W2)
    y1, z, yu = _node_mm(x, s2, W1, W3, U1)
    p2 = _msg_segsum(y1, z, a2, src, dst)
    return _final_mm(yu, p2, U2)


# speedup vs baseline: 3.4746x; 1.1522x over previous
import functools

import jax
import jax.numpy as jnp
from jax import lax
from jax.experimental import pallas as pl
from jax.experimental.pallas import tpu as pltpu
from jax.experimental.pallas import tpu_sc as plsc

N = 10000
E = 320000
D = 64
NC = 2
NS = 16
NW = NC * NS
EPT = E // NW
NPT_BASE = 624
NPT_LAST = N - (NS - 1) * NPT_BASE


def _mesh():
    return plsc.VectorSubcoreMesh(core_axis_name="c", subcore_axis_name="s")


def _zero_rows(buf, nrows):
    zero = jnp.zeros((16,), jnp.float32)

    def body(i, _):
        for j in range(D // 16):
            buf[i, pl.ds(j * 16, 16)] = zero
        return 0

    lax.fori_loop(0, nrows, body, 0, unroll=False)


def _zero_acc_slab(buf, acc_sh, s, cap):
    row0 = s * NPT_BASE
    _zero_rows(buf, min(cap, NPT_LAST))

    def copy_rows(total):
        off = 0
        while off < total:
            h = min(cap, total - off)
            pltpu.sync_copy(buf.at[pl.ds(0, h)], acc_sh.at[pl.ds(row0 + off, h)])
            off += h

    @pl.when(s == NS - 1)
    def _():
        copy_rows(NPT_LAST)

    @pl.when(s != NS - 1)
    def _():
        copy_rows(NPT_BASE)


def _dump_acc_slab(acc_sh, out_hbm, buf, c, s, cap):
    row0 = s * NPT_BASE

    def copy_rows(total):
        off = 0
        while off < total:
            h = min(cap, total - off)
            pltpu.sync_copy(acc_sh.at[pl.ds(row0 + off, h)], buf.at[pl.ds(0, h)])
            pltpu.sync_copy(buf.at[pl.ds(0, h)], out_hbm.at[c, pl.ds(row0 + off, h)])
            off += h

    @pl.when(s == NS - 1)
    def _():
        copy_rows(NPT_LAST)

    @pl.when(s != NS - 1)
    def _():
        copy_rows(NPT_BASE)


CB_A = 400


def _segsum_body(eh_hbm, src_hbm, out_hbm,
                  rows0, rows1, idx0, idx1, acc_sh,
                  ld0, ld1, sc0, sc1):
    c = lax.axis_index("c")
    s = lax.axis_index("s")

    _zero_acc_slab(rows0, acc_sh, s, CB_A)
    plsc.subcore_barrier()

    rows = (rows0, rows1)
    idx = (idx0, idx1)
    ld = (ld0, ld1)
    sc = (sc0, sc1)
    base0 = (c * NS + s) * EPT
    nchunks = EPT // CB_A

    pend_ld = [None, None]
    pend_sc = [None, None]

    def issue_loads(k, b):
        base = base0 + k * CB_A
        d1 = pltpu.async_copy(src_hbm.at[pl.ds(base, CB_A)], idx[b], ld[b])
        d2 = pltpu.async_copy(eh_hbm.at[pl.ds(base, CB_A)], rows[b], ld[b])
        pend_ld[b] = (d1, d2)

    issue_loads(0, 0)
    for k in range(nchunks):
        cur = k & 1
        oth = 1 - cur
        for d in pend_ld[cur]:
            d.wait()
        if k + 1 < nchunks:
            if pend_sc[oth] is not None:
                pend_sc[oth].wait()
            issue_loads(k + 1, oth)
        pend_sc[cur] = pltpu.async_copy(rows[cur], acc_sh.at[idx[cur]],
                                        sc[cur], add=True)

    for b in (0, 1):
        if pend_sc[b] is not None:
            pend_sc[b].wait()

    plsc.subcore_barrier()
    _dump_acc_slab(acc_sh, out_hbm, rows0, c, s, CB_A)


def _segsum_eh(edge_hidden, src):
    k = functools.partial(
        pl.kernel,
        out_type=jax.ShapeDtypeStruct((NC, N, D), jnp.float32),
        mesh=_mesh(),
        compiler_params=pltpu.CompilerParams(use_tc_tiling_on_sc=False),
        scratch_types=[
            pltpu.VMEM((CB_A, D), jnp.float32),
            pltpu.VMEM((CB_A, D), jnp.float32),
            pltpu.VMEM((CB_A,), jnp.int32),
            pltpu.VMEM((CB_A,), jnp.int32),
            pltpu.VMEM_SHARED((N, D), jnp.float32),
            pltpu.SemaphoreType.DMA,
            pltpu.SemaphoreType.DMA,
            pltpu.SemaphoreType.DMA,
            pltpu.SemaphoreType.DMA,
        ],
    )(_segsum_body)
    return k(edge_hidden, src)


CB_C = 200


def _msg_body(y1_hbm, z_hbm, a2_hbm, src_hbm, dst_hbm, out_hbm,
               s0, s1, d0, d1, ys0, ys1, zd0, zd1, a20, a21, acc_sh,
               li0, li1, lg0, lg1, la0, la1, sc0, sc1):
    c = lax.axis_index("c")
    s = lax.axis_index("s")

    _zero_acc_slab(ys0, acc_sh, s, CB_C)
    plsc.subcore_barrier()

    idx_s = (s0, s1)
    idx_d = (d0, d1)
    ys = (ys0, ys1)
    zd = (zd0, zd1)
    a2 = (a20, a21)
    li = (li0, li1)
    lg = (lg0, lg1)
    la = (la0, la1)
    sc = (sc0, sc1)

    base0 = (c * NS + s) * EPT
    nchunks = EPT // CB_C

    pend_idx = [None, None]
    pend_g = [None, None]
    pend_a2 = [None, None]
    pend_sc = [None, None]

    def issue_idx_a2(k, b):
        base = base0 + k * CB_C
        d1_ = pltpu.async_copy(src_hbm.at[pl.ds(base, CB_C)], idx_s[b], li[b])
        d2_ = pltpu.async_copy(dst_hbm.at[pl.ds(base, CB_C)], idx_d[b], li[b])
        d3_ = pltpu.async_copy(a2_hbm.at[pl.ds(base, CB_C)], a2[b], la[b])
        pend_idx[b] = (d1_, d2_)
        pend_a2[b] = d3_

    def issue_gathers(b):
        g1 = pltpu.async_copy(y1_hbm.at[idx_s[b]], ys[b], lg[b])
        g2 = pltpu.async_copy(z_hbm.at[idx_d[b]], zd[b], lg[b])
        pend_g[b] = (g1, g2)

    def compute(b):
        ab, yb, zb = a2[b], ys[b], zd[b]

        def body(i, _):
            for j in range(D // 16):
                sl = pl.ds(j * 16, 16)
                v = ab[i, sl] + yb[i, sl] + zb[i, sl]
                ab[i, sl] = jnp.maximum(v, 0.0)
            return 0

        lax.fori_loop(0, CB_C, body, 0, unroll=False)

    issue_idx_a2(0, 0)
    for d_ in pend_idx[0]:
        d_.wait()
    issue_gathers(0)

    for k in range(nchunks):
        cur = k & 1
        oth = 1 - cur
        if k + 1 < nchunks:
            if pend_sc[oth] is not None:
                pend_sc[oth].wait()
            issue_idx_a2(k + 1, oth)
        for d_ in pend_g[cur]:
            d_.wait()
        pend_a2[cur].wait()
        if k + 1 < nchunks:
            for d_ in pend_idx[oth]:
                d_.wait()
            issue_gathers(oth)
        compute(cur)
        pend_sc[cur] = pltpu.async_copy(a2[cur], acc_sh.at[idx_s[cur]],
                                        sc[cur], add=True)

    for b in (0, 1):
        if pend_sc[b] is not None:
            pend_sc[b].wait()

    plsc.subcore_barrier()
    _dump_acc_slab(acc_sh, out_hbm, ys0, c, s, CB_C)


def _msg_segsum(y1, z, a2, src, dst):
    k = functools.partial(
        pl.kernel,
        out_type=jax.ShapeDtypeStruct((NC, N, D), jnp.float32),
        mesh=_mesh(),
        compiler_params=pltpu.CompilerParams(use_tc_tiling_on_sc=False),
        scratch_types=[
            pltpu.VMEM((CB_C,), jnp.int32),
            pltpu.VMEM((CB_C,), jnp.int32),
            pltpu.VMEM((CB_C,), jnp.int32),
            pltpu.VMEM((CB_C,), jnp.int32),
            pltpu.VMEM((CB_C, D), jnp.float32),
            pltpu.VMEM((CB_C, D), jnp.float32),
            pltpu.VMEM((CB_C, D), jnp.float32),
            pltpu.VMEM((CB_C, D), jnp.float32),
            pltpu.VMEM((CB_C, D), jnp.float32),
            pltpu.VMEM((CB_C, D), jnp.float32),
            pltpu.VMEM_SHARED((N, D), jnp.float32),
            pltpu.SemaphoreType.DMA,
            pltpu.SemaphoreType.DMA,
            pltpu.SemaphoreType.DMA,
            pltpu.SemaphoreType.DMA,
            pltpu.SemaphoreType.DMA,
            pltpu.SemaphoreType.DMA,
            pltpu.SemaphoreType.DMA,
            pltpu.SemaphoreType.DMA,
        ],
    )(_msg_body)
    return k(y1, z, a2, src, dst)


_DN = (((1,), (1,)), ((), ()))


def _node_mm_kernel(x_ref, s2_ref, w1_ref, w3_ref, u1_ref, y1_ref, z_ref, yu_ref):
    xb = x_ref[...]
    y1_ref[...] = lax.dot_general(xb, w1_ref[...], _DN,
                                  preferred_element_type=jnp.float32)
    yu_ref[...] = lax.dot_general(xb, u1_ref[...], _DN,
                                  preferred_element_type=jnp.float32)
    sb = s2_ref[0] + s2_ref[1]
    z_ref[...] = lax.dot_general(sb, w3_ref[...], _DN,
                                 preferred_element_type=jnp.float32)


def _node_mm(x, s2, w1, w3, u1):
    bn = 1000
    grid = (N // bn,)
    return pl.pallas_call(
        _node_mm_kernel,
        grid=grid,
        in_specs=[
            pl.BlockSpec((bn, 128), lambda i: (i, 0)),
            pl.BlockSpec((NC, bn, D), lambda i: (0, i, 0)),
            pl.BlockSpec((D, 128), lambda i: (0, 0)),
            pl.BlockSpec((D, D), lambda i: (0, 0)),
            pl.BlockSpec((D, 128), lambda i: (0, 0)),
        ],
        out_specs=[
            pl.BlockSpec((bn, D), lambda i: (i, 0)),
            pl.BlockSpec((bn, D), lambda i: (i, 0)),
            pl.BlockSpec((bn, D), lambda i: (i, 0)),
        ],
        out_shape=[jax.ShapeDtypeStruct((N, D), jnp.float32)] * 3,
    )(x, s2, w1, w3, u1)


def _edge_mm_kernel(ea_ref, w2_ref, a2_ref):
    a2_ref[...] = lax.dot_general(ea_ref[...], w2_ref[...], _DN,
                                  preferred_element_type=jnp.float32)


def _edge_mm(edge_attr, w2):
    be = 8000
    grid = (E // be,)
    return pl.pallas_call(
        _edge_mm_kernel,
        grid=grid,
        in_specs=[
            pl.BlockSpec((be, 16), lambda i: (i, 0)),
            pl.BlockSpec((D, 16), lambda i: (0, 0)),
        ],
        out_specs=pl.BlockSpec((be, D), lambda i: (i, 0)),
        out_shape=jax.ShapeDtypeStruct((E, D), jnp.float32),
    )(edge_attr, w2)


def _final_mm_kernel(yu_ref, p2_ref, u2_ref, out_ref):
    pb = p2_ref[0] + p2_ref[1]
    out_ref[...] = yu_ref[...] + lax.dot_general(
        pb, u2_ref[...], _DN, preferred_element_type=jnp.float32)


def _final_mm(yu, p2, u2):
    bn = 1000
    grid = (N // bn,)
    return pl.pallas_call(
        _final_mm_kernel,
        grid=grid,
        in_specs=[
            pl.BlockSpec((bn, D), lambda i: (i, 0)),
            pl.BlockSpec((NC, bn, D), lambda i: (0, i, 0)),
            pl.BlockSpec((D, D), lambda i: (0, 0)),
        ],
        out_specs=pl.BlockSpec((bn, D), lambda i: (i, 0)),
        out_shape=jax.ShapeDtypeStruct((N, D), jnp.float32),
    )(yu, p2, u2)


def kernel(x, edge_attr, edge_hidden, W1, W2, W3, U1, U2, edge_index):
    src = edge_index[0]
    dst = edge_index[1]

    s2 = _segsum_eh(edge_hidden, src)
    a2 = _edge_mm(edge_attr, W2)
    y1, z, yu = _node_mm(x, s2, W1, W3, U1)
    p2 = _msg_segsum(y1, z, a2, src, dst)
    return _final_mm(yu, p2, U2)
